# baseline (device time: 19903 ns/iter reference)
import jax
import jax.numpy as jnp
from jax import lax
from jax.experimental import pallas as pl
from jax.experimental.pallas import tpu as pltpu

N_DEV = 4
M_PER = 8192
N = 1024
BLOCK_M = 2048
N_BLOCKS = M_PER // BLOCK_M
N_COLS = 2
BLOCK_N = N // N_COLS


def _body(x_ref, out_ref, acc_ref, comm_ref, send_sems, recv_sems):
    c = pl.program_id(0)
    r = pl.program_id(1)
    my_pos = lax.axis_index("i")
    barrier_sem = pltpu.get_barrier_semaphore()

    @pl.when((c == 0) & (r == 0))
    def _():
        for d in range(1, N_DEV):
            pl.semaphore_signal(
                barrier_sem,
                inc=1,
                device_id=((my_pos + d) % N_DEV,),
                device_id_type=pl.DeviceIdType.MESH,
            )

    xb = x_ref[:, :]
    bmax = jnp.max(xb, axis=0, keepdims=True)
    rows = lax.broadcasted_iota(jnp.int32, (BLOCK_M, BLOCK_N), 0)
    bidx = jnp.min(jnp.where(xb == bmax, rows, BLOCK_M), axis=0, keepdims=True)
    base = my_pos * M_PER + r * BLOCK_M
    gidx = (base + bidx).astype(jnp.float32)

    @pl.when(r == 0)
    def _():
        acc_ref[0:1, :] = bmax
        acc_ref[1:2, :] = gidx

    @pl.when(r > 0)
    def _():
        rv = acc_ref[0:1, :]
        ri = acc_ref[1:2, :]
        better = bmax > rv
        acc_ref[0:1, :] = jnp.where(better, bmax, rv)
        acc_ref[1:2, :] = jnp.where(better, gidx, ri)

    def send_half(ch):
        comm_ref[ch, pl.ds(my_pos, 1)] = acc_ref[:, :].reshape(1, 2, BLOCK_N)
        sends = []
        for d in range(1, N_DEV):
            peer = (my_pos + d) % N_DEV
            rdma = pltpu.make_async_remote_copy(
                src_ref=comm_ref.at[ch, my_pos],
                dst_ref=comm_ref.at[ch, my_pos],
                send_sem=send_sems.at[ch, d],
                recv_sem=recv_sems.at[ch, my_pos],
                device_id=(peer,),
                device_id_type=pl.DeviceIdType.MESH,
            )
            rdma.start()
            sends.append(rdma)
        return sends

    def wait_and_reduce_half(ch):
        for d in range(1, N_DEV):
            peer = (my_pos + d) % N_DEV
            recv = pltpu.make_async_remote_copy(
                src_ref=comm_ref.at[ch, peer],
                dst_ref=comm_ref.at[ch, peer],
                send_sem=send_sems.at[ch, d],
                recv_sem=recv_sems.at[ch, peer],
                device_id=(peer,),
                device_id_type=pl.DeviceIdType.MESH,
            )
            recv.wait_recv()
        rv = comm_ref[ch, 0, 0:1, :]
        ri = comm_ref[ch, 0, 1:2, :]
        for k in range(1, N_DEV):
            v = comm_ref[ch, k, 0:1, :]
            i = comm_ref[ch, k, 1:2, :]
            better = v > rv
            rv = jnp.where(better, v, rv)
            ri = jnp.where(better, i, ri)
        out_ref[0:1, pl.ds(ch * BLOCK_N, BLOCK_N)] = rv
        out_ref[1:2, pl.ds(ch * BLOCK_N, BLOCK_N)] = ri

    @pl.when((c == 0) & (r == N_BLOCKS - 1))
    def _():
        pl.semaphore_wait(barrier_sem, N_DEV - 1)
        send_half(0)

    @pl.when((c == 1) & (r == N_BLOCKS - 1))
    def _():
        sends = send_half(1)
        wait_and_reduce_half(0)
        wait_and_reduce_half(1)
        for d in range(1, N_DEV):
            peer = (my_pos + d) % N_DEV
            done = pltpu.make_async_remote_copy(
                src_ref=comm_ref.at[0, my_pos],
                dst_ref=comm_ref.at[0, my_pos],
                send_sem=send_sems.at[0, d],
                recv_sem=recv_sems.at[0, my_pos],
                device_id=(peer,),
                device_id_type=pl.DeviceIdType.MESH,
            )
            done.wait_send()
        for rdma in sends:
            rdma.wait_send()


def kernel(x):
    return pl.pallas_call(
        _body,
        grid=(N_COLS, N_BLOCKS),
        in_specs=[pl.BlockSpec((BLOCK_M, BLOCK_N), lambda c, r: (r, c))],
        out_specs=pl.BlockSpec((2, N), lambda c, r: (0, 0)),
        out_shape=jax.ShapeDtypeStruct((2, N), jnp.float32),
        scratch_shapes=[
            pltpu.VMEM((2, BLOCK_N), jnp.float32),
            pltpu.VMEM((N_COLS, N_DEV, 2, BLOCK_N), jnp.float32),
            pltpu.SemaphoreType.DMA((N_COLS, N_DEV)),
            pltpu.SemaphoreType.DMA((N_COLS, N_DEV)),
        ],
        compiler_params=pltpu.CompilerParams(collective_id=0),
    )(x)


# device time: 17980 ns/iter; 1.1070x vs baseline; 1.1070x over previous
import jax
import jax.numpy as jnp
from jax import lax
from jax.experimental import pallas as pl
from jax.experimental.pallas import tpu as pltpu

N_DEV = 4
M_PER = 8192
N = 1024
BLOCK_M = 2048
N_BLOCKS = M_PER // BLOCK_M


def _body(x_ref, out_ref, acc_ref, comm_ref, send_sems, recv_sems):
    pid = pl.program_id(0)
    my_pos = lax.axis_index("i")
    barrier_sem = pltpu.get_barrier_semaphore()

    @pl.when(pid == 0)
    def _():
        for d in range(1, N_DEV):
            pl.semaphore_signal(
                barrier_sem,
                inc=1,
                device_id=((my_pos + d) % N_DEV,),
                device_id_type=pl.DeviceIdType.MESH,
            )

    xb = x_ref[:, :]
    bmax = jnp.max(xb, axis=0, keepdims=True)
    rows = lax.broadcasted_iota(jnp.int32, (BLOCK_M, N), 0)
    bidx = jnp.min(jnp.where(xb == bmax, rows, BLOCK_M), axis=0, keepdims=True)
    base = my_pos * M_PER + pid * BLOCK_M
    gidx = (base + bidx).astype(jnp.float32)

    @pl.when(pid == 0)
    def _():
        acc_ref[0:1, :] = bmax
        acc_ref[1:2, :] = gidx

    @pl.when((pid > 0) & (pid < N_BLOCKS - 1))
    def _():
        rv = acc_ref[0:1, :]
        ri = acc_ref[1:2, :]
        better = bmax > rv
        acc_ref[0:1, :] = jnp.where(better, bmax, rv)
        acc_ref[1:2, :] = jnp.where(better, gidx, ri)

    @pl.when(pid == N_BLOCKS - 1)
    def _():
        rv = acc_ref[0:1, :]
        ri = acc_ref[1:2, :]
        better = bmax > rv
        comm_ref[pl.ds(my_pos, 1), 0:1, :] = jnp.where(better, bmax, rv).reshape(
            1, 1, N
        )
        comm_ref[pl.ds(my_pos, 1), 1:2, :] = jnp.where(better, gidx, ri).reshape(
            1, 1, N
        )

        pl.semaphore_wait(barrier_sem, N_DEV - 1)
        sends = []
        for d in range(1, N_DEV):
            peer = (my_pos + d) % N_DEV
            rdma = pltpu.make_async_remote_copy(
                src_ref=comm_ref.at[my_pos],
                dst_ref=comm_ref.at[my_pos],
                send_sem=send_sems.at[d],
                recv_sem=recv_sems.at[my_pos],
                device_id=(peer,),
                device_id_type=pl.DeviceIdType.MESH,
            )
            rdma.start()
            sends.append(rdma)

        rv = comm_ref[my_pos, 0:1, :]
        ri = comm_ref[my_pos, 1:2, :]
        for d in (1, 3, 2):
            peer = (my_pos + d) % N_DEV
            recv = pltpu.make_async_remote_copy(
                src_ref=comm_ref.at[peer],
                dst_ref=comm_ref.at[peer],
                send_sem=send_sems.at[d],
                recv_sem=recv_sems.at[peer],
                device_id=(peer,),
                device_id_type=pl.DeviceIdType.MESH,
            )
            recv.wait_recv()
            v = comm_ref[peer, 0:1, :]
            i = comm_ref[peer, 1:2, :]
            take = (v > rv) | ((v == rv) & (i < ri))
            rv = jnp.where(take, v, rv)
            ri = jnp.where(take, i, ri)
        out_ref[0:1, :] = rv
        out_ref[1:2, :] = ri
        for rdma in sends:
            rdma.wait_send()


def kernel(x):
    return pl.pallas_call(
        _body,
        grid=(N_BLOCKS,),
        in_specs=[pl.BlockSpec((BLOCK_M, N), lambda i: (i, 0))],
        out_specs=pl.BlockSpec((2, N), lambda i: (0, 0)),
        out_shape=jax.ShapeDtypeStruct((2, N), jnp.float32),
        scratch_shapes=[
            pltpu.VMEM((2, N), jnp.float32),
            pltpu.VMEM((N_DEV, 2, N), jnp.float32),
            pltpu.SemaphoreType.DMA((N_DEV,)),
            pltpu.SemaphoreType.DMA((N_DEV,)),
        ],
        compiler_params=pltpu.CompilerParams(collective_id=0),
    )(x)


# device time: 17958 ns/iter; 1.1083x vs baseline; 1.0012x over previous
import jax
import jax.numpy as jnp
from jax import lax
from jax.experimental import pallas as pl
from jax.experimental.pallas import tpu as pltpu

N_DEV = 4
M_PER = 8192
N = 1024
BLOCK_M = 2048
N_BLOCKS = M_PER // BLOCK_M


def _body(x_ref, out_ref, acc_ref, comm_ref, send_sems, recv_sems):
    pid = pl.program_id(0)
    my_pos = lax.axis_index("i")
    barrier_sem = pltpu.get_barrier_semaphore()

    @pl.when(pid == 0)
    def _():
        for d in range(1, N_DEV):
            pl.semaphore_signal(
                barrier_sem,
                inc=1,
                device_id=((my_pos + d) % N_DEV,),
                device_id_type=pl.DeviceIdType.MESH,
            )

    xb = x_ref[:, :]
    bmax = jnp.max(xb, axis=0, keepdims=True)
    rows = lax.broadcasted_iota(jnp.int32, (BLOCK_M, N), 0)
    bidx = jnp.min(jnp.where(xb == bmax, rows, BLOCK_M), axis=0, keepdims=True)
    base = my_pos * M_PER + pid * BLOCK_M
    gidx = (base + bidx).astype(jnp.float32)

    @pl.when(pid == 0)
    def _():
        acc_ref[0:1, :] = bmax
        acc_ref[1:2, :] = gidx

    @pl.when((pid > 0) & (pid < N_BLOCKS - 1))
    def _():
        rv = acc_ref[0:1, :]
        ri = acc_ref[1:2, :]
        better = bmax > rv
        acc_ref[0:1, :] = jnp.where(better, bmax, rv)
        acc_ref[1:2, :] = jnp.where(better, gidx, ri)

    @pl.when(pid == N_BLOCKS - 2)
    def _():
        pl.semaphore_wait(barrier_sem, N_DEV - 1)

    @pl.when(pid == N_BLOCKS - 1)
    def _():
        rv = acc_ref[0:1, :]
        ri = acc_ref[1:2, :]
        better = bmax > rv
        comm_ref[pl.ds(my_pos, 1), 0:1, :] = jnp.where(better, bmax, rv).reshape(
            1, 1, N
        )
        comm_ref[pl.ds(my_pos, 1), 1:2, :] = jnp.where(better, gidx, ri).reshape(
            1, 1, N
        )

        sends = []
        for d in range(1, N_DEV):
            peer = (my_pos + d) % N_DEV
            rdma = pltpu.make_async_remote_copy(
                src_ref=comm_ref.at[my_pos],
                dst_ref=comm_ref.at[my_pos],
                send_sem=send_sems.at[d],
                recv_sem=recv_sems.at[my_pos],
                device_id=(peer,),
                device_id_type=pl.DeviceIdType.MESH,
            )
            rdma.start()
            sends.append(rdma)

        rv = comm_ref[my_pos, 0:1, :]
        ri = comm_ref[my_pos, 1:2, :]
        for d in (1, 3, 2):
            peer = (my_pos + d) % N_DEV
            recv = pltpu.make_async_remote_copy(
                src_ref=comm_ref.at[peer],
                dst_ref=comm_ref.at[peer],
                send_sem=send_sems.at[d],
                recv_sem=recv_sems.at[peer],
                device_id=(peer,),
                device_id_type=pl.DeviceIdType.MESH,
            )
            recv.wait_recv()
            v = comm_ref[peer, 0:1, :]
            i = comm_ref[peer, 1:2, :]
            take = (v > rv) | ((v == rv) & (i < ri))
            rv = jnp.where(take, v, rv)
            ri = jnp.where(take, i, ri)
        out_ref[0:1, :] = rv
        out_ref[1:2, :] = ri
        for rdma in sends:
            rdma.wait_send()


def kernel(x):
    return pl.pallas_call(
        _body,
        grid=(N_BLOCKS,),
        in_specs=[pl.BlockSpec((BLOCK_M, N), lambda i: (i, 0))],
        out_specs=pl.BlockSpec((2, N), lambda i: (0, 0)),
        out_shape=jax.ShapeDtypeStruct((2, N), jnp.float32),
        scratch_shapes=[
            pltpu.VMEM((2, N), jnp.float32),
            pltpu.VMEM((N_DEV, 2, N), jnp.float32),
            pltpu.SemaphoreType.DMA((N_DEV,)),
            pltpu.SemaphoreType.DMA((N_DEV,)),
        ],
        compiler_params=pltpu.CompilerParams(collective_id=0),
    )(x)


# device time: 17059 ns/iter; 1.1667x vs baseline; 1.0527x over previous
import jax
import jax.numpy as jnp
from jax import lax
from jax.experimental import pallas as pl
from jax.experimental.pallas import tpu as pltpu

N_DEV = 4
M_PER = 8192
N = 1024
CHUNK_M = 512
N_CHUNKS = M_PER // CHUNK_M
N_BUF = 6


def _body(x_hbm, out_ref, buf, copy_sems, comm_ref, send_sems, recv_sems):
    my_pos = lax.axis_index("i")
    barrier_sem = pltpu.get_barrier_semaphore()

    for d in range(1, N_DEV):
        pl.semaphore_signal(
            barrier_sem,
            inc=1,
            device_id=((my_pos + d) % N_DEV,),
            device_id_type=pl.DeviceIdType.MESH,
        )

    def copy(k):
        return pltpu.make_async_copy(
            x_hbm.at[pl.ds(k * CHUNK_M, CHUNK_M), :],
            buf.at[k % N_BUF],
            copy_sems.at[k % N_BUF],
        )

    for k in range(N_BUF - 1):
        copy(k).start()

    rows = lax.broadcasted_iota(jnp.int32, (CHUNK_M, N), 0)
    rv = None
    ri = None
    for k in range(N_CHUNKS):
        copy(k).wait()
        xb = buf[k % N_BUF]
        bmax = jnp.max(xb, axis=0, keepdims=True)
        bidx = jnp.min(
            jnp.where(xb == bmax, rows, CHUNK_M), axis=0, keepdims=True
        )
        gidx = (my_pos * M_PER + k * CHUNK_M + bidx).astype(jnp.float32)
        if k + N_BUF - 1 < N_CHUNKS:
            copy(k + N_BUF - 1).start()
        if rv is None:
            rv, ri = bmax, gidx
        else:
            better = bmax > rv
            rv = jnp.where(better, bmax, rv)
            ri = jnp.where(better, gidx, ri)

    comm_ref[pl.ds(my_pos, 1), 0:1, :] = rv.reshape(1, 1, N)
    comm_ref[pl.ds(my_pos, 1), 1:2, :] = ri.reshape(1, 1, N)
    pl.semaphore_wait(barrier_sem, N_DEV - 1)
    sends = []
    for d in range(1, N_DEV):
        peer = (my_pos + d) % N_DEV
        rdma = pltpu.make_async_remote_copy(
            src_ref=comm_ref.at[my_pos],
            dst_ref=comm_ref.at[my_pos],
            send_sem=send_sems.at[d],
            recv_sem=recv_sems.at[my_pos],
            device_id=(peer,),
            device_id_type=pl.DeviceIdType.MESH,
        )
        rdma.start()
        sends.append(rdma)

    for d in (1, 3, 2):
        peer = (my_pos + d) % N_DEV
        recv = pltpu.make_async_remote_copy(
            src_ref=comm_ref.at[peer],
            dst_ref=comm_ref.at[peer],
            send_sem=send_sems.at[d],
            recv_sem=recv_sems.at[peer],
            device_id=(peer,),
            device_id_type=pl.DeviceIdType.MESH,
        )
        recv.wait_recv()
        v = comm_ref[peer, 0:1, :]
        i = comm_ref[peer, 1:2, :]
        take = (v > rv) | ((v == rv) & (i < ri))
        rv = jnp.where(take, v, rv)
        ri = jnp.where(take, i, ri)
    out_ref[0:1, :] = rv
    out_ref[1:2, :] = ri
    for rdma in sends:
        rdma.wait_send()


def kernel(x):
    return pl.pallas_call(
        _body,
        in_specs=[pl.BlockSpec(memory_space=pl.ANY)],
        out_specs=pl.BlockSpec(memory_space=pltpu.VMEM),
        out_shape=jax.ShapeDtypeStruct((2, N), jnp.float32),
        scratch_shapes=[
            pltpu.VMEM((N_BUF, CHUNK_M, N), jnp.float32),
            pltpu.SemaphoreType.DMA((N_BUF,)),
            pltpu.VMEM((N_DEV, 2, N), jnp.float32),
            pltpu.SemaphoreType.DMA((N_DEV,)),
            pltpu.SemaphoreType.DMA((N_DEV,)),
        ],
        compiler_params=pltpu.CompilerParams(
            collective_id=0,
            vmem_limit_bytes=50 * 1024 * 1024,
        ),
    )(x)
